# baseline (speedup 1.0000x reference)
import functools

import jax
import jax.numpy as jnp
from jax import lax
from jax.experimental import pallas as pl
from jax.experimental.pallas import tpu as pltpu
from jax.experimental.pallas import tpu_sc as plsc

N, P, NBR, D_IN, D_OUT = 1024, 1024, 8, 256, 256

BI = 16
NSTEP = N // BI
NC, NS, L = 2, 16, 16
NW = NC * NS
SPW = N // NW
ROWS = SPW * NBR
CHUNK = 128
NCHUNK = ROWS // CHUNK


def _tc_kernel(xnss_ref, x_ref, w_ref, b_ref, idx_ref, y_ref):
    i = pl.program_id(0)

    @pl.when(i == 0)
    def _matmul():
        y_ref[...] = (
            lax.dot_general(
                x_ref[...], w_ref[...],
                (((1,), (1,)), ((), ())),
                preferred_element_type=jnp.float32,
                precision=lax.Precision.HIGHEST,
            )
            + b_ref[...]
        )

    blk = xnss_ref[...].reshape(BI, BI * NBR)
    row = lax.broadcasted_iota(jnp.int32, (BI, BI * NBR), 0)
    col = lax.broadcasted_iota(jnp.int32, (BI, BI * NBR), 1)
    mask = (col // NBR) == row
    diag = jnp.sum(jnp.where(mask, blk, 0), axis=0)
    idx_ref[pl.ds(i % 8, 1), :] = diag.reshape(1, BI * NBR)


def _sc_body(idx_hbm, y_hbm, out_hbm, idx_vm, rows_v, sem):
    wid = lax.axis_index("s") * NC + lax.axis_index("c")
    pltpu.sync_copy(idx_hbm.at[pl.ds(wid * NCHUNK, NCHUNK)], idx_vm)
    cps = [
        pltpu.async_copy(
            y_hbm.at[idx_vm.at[c]], rows_v.at[pl.ds(c * CHUNK, CHUNK)], sem
        )
        for c in range(NCHUNK)
    ]
    for cp in cps:
        cp.wait()
    pltpu.sync_copy(rows_v, out_hbm.at[pl.ds(wid * ROWS, ROWS)])


def kernel(X_sites, X_NSs, N_sites, W, b):
    idx, y = pl.pallas_call(
        _tc_kernel,
        grid=(NSTEP,),
        in_specs=[
            pl.BlockSpec((BI, BI, NBR), lambda i: (i, i, 0)),
            pl.BlockSpec((N, D_IN), lambda i: (0, 0)),
            pl.BlockSpec((D_OUT, D_IN), lambda i: (0, 0)),
            pl.BlockSpec((1, D_OUT), lambda i: (0, 0)),
        ],
        out_specs=[
            pl.BlockSpec((8, 128), lambda i: (i // 8, 0)),
            pl.BlockSpec((N, D_OUT), lambda i: (0, 0)),
        ],
        out_shape=[
            jax.ShapeDtypeStruct((N * NBR // 128, 128), jnp.int32),
            jax.ShapeDtypeStruct((N, D_OUT), jnp.float32),
        ],
    )(X_NSs, X_sites, W, b.reshape(1, D_OUT))

    mesh = plsc.VectorSubcoreMesh(core_axis_name="c", subcore_axis_name="s")
    out = pl.kernel(
        _sc_body,
        mesh=mesh,
        out_type=jax.ShapeDtypeStruct((N * NBR, D_OUT), jnp.float32),
        scratch_types=[
            pltpu.VMEM((NCHUNK, CHUNK), jnp.int32),
            pltpu.VMEM((ROWS, D_OUT), jnp.float32),
            pltpu.SemaphoreType.DMA,
        ],
    )(idx, y)
    return out.reshape(N, NBR, D_OUT)
